# baseline (speedup 1.0000x reference)
import functools

import jax
import jax.numpy as jnp
import numpy as np
from jax import lax
from jax.experimental import pallas as pl
from jax.experimental.pallas import tpu as pltpu
from jax.experimental.pallas import tpu_sc as plsc

_B, _L = 16, 2048
_CH = 16
_SEG = _L // _CH
_IMAX = np.int32(2147483647)
_NEG_INF = np.float32(-np.inf)


def _body(start_hbm, end_hbm, score_out, i_out, j_out,
          start_v, end_v, seg_s, score_s, i_s, j_s):
    w = lax.axis_index("s") * 2 + lax.axis_index("c")

    @pl.when(w < _B)
    def _():
        row = w
        pltpu.sync_copy(start_hbm.at[row], start_v)
        pltpu.sync_copy(end_hbm.at[row], end_v)

        lane = lax.iota(jnp.int32, _CH)
        lanebase = lane * _SEG

        def segmax(k, m):
            e = plsc.load_gather(end_v, [lanebase + k])
            return jnp.maximum(m, e)

        sm = lax.fori_loop(0, _SEG, segmax,
                           jnp.zeros((_CH,), jnp.float32) + _NEG_INF,
                           unroll=8)

        segsuf = lax.rev(plsc.cummax(lax.rev(sm, (0,))), (0,))
        seg_s[...] = segsuf
        h = plsc.load_gather(seg_s, [jnp.minimum(lane + 1, _CH - 1)])
        h = jnp.where(lane == _CH - 1, _NEG_INF, h)

        def bwd(k, state):
            mm, bestv, besti = state
            kk = _SEG - 1 - k
            e = plsc.load_gather(end_v, [lanebase + kk])
            st = plsc.load_gather(start_v, [lanebase + kk])
            mm = jnp.maximum(mm, e)
            c = st + jnp.maximum(mm, h)
            upd = c >= bestv
            bestv = jnp.where(upd, c, bestv)
            besti = jnp.where(upd, lanebase + kk, besti)
            return mm, bestv, besti

        init = (jnp.zeros((_CH,), jnp.float32) + _NEG_INF,
                jnp.zeros((_CH,), jnp.float32) + _NEG_INF,
                jnp.zeros((_CH,), jnp.int32))
        _, bestv, besti = lax.fori_loop(0, _SEG, bwd, init, unroll=8)

        best = jnp.max(bestv)
        istar = jnp.min(jnp.where(bestv == best, besti, _IMAX))
        lstar = istar // _SEG

        wacc = jnp.zeros((_CH,), jnp.float32) + _NEG_INF
        sbase = lstar * _SEG
        for t in range(_SEG // _CH):
            base = pl.multiple_of(sbase + t * _CH, _CH)
            e = end_v[pl.ds(base, _CH)]
            idxv = lane + base
            wacc = jnp.maximum(wacc, jnp.where(idxv >= istar, e, _NEG_INF))
        wval = jnp.max(wacc)
        hval = jnp.max(jnp.where(lane == lstar, h, _NEG_INF))
        target = jnp.maximum(wval, hval)
        inseg = wval >= hval

        lnext = jnp.min(jnp.where((lane > lstar) & (sm == target), lane, _IMAX))
        lsel = jnp.where(inseg, lstar, lnext)
        jlo = jnp.where(inseg, istar, lsel * _SEG)
        jacc = jnp.zeros((_CH,), jnp.int32) + _IMAX
        jbase = lsel * _SEG
        for t in range(_SEG // _CH):
            base = pl.multiple_of(jbase + t * _CH, _CH)
            e = end_v[pl.ds(base, _CH)]
            idxv = lane + base
            m = (idxv >= jlo) & (e == target)
            jacc = jnp.minimum(jacc, jnp.where(m, idxv, _IMAX))
        jstar = jnp.min(jacc)

        score_s[...] = jnp.zeros((_CH,), jnp.float32) + best
        i_s[...] = jnp.zeros((_CH,), jnp.int32) + istar
        j_s[...] = jnp.zeros((_CH,), jnp.int32) + jstar
        pltpu.sync_copy(score_s, score_out.at[row])
        pltpu.sync_copy(i_s, i_out.at[row])
        pltpu.sync_copy(j_s, j_out.at[row])


_sc_call = functools.partial(
    pl.kernel,
    mesh=plsc.VectorSubcoreMesh(core_axis_name="c", subcore_axis_name="s"),
    compiler_params=pltpu.CompilerParams(needs_layout_passes=False),
    out_type=[
        jax.ShapeDtypeStruct((_B, _CH), jnp.float32),
        jax.ShapeDtypeStruct((_B, _CH), jnp.int32),
        jax.ShapeDtypeStruct((_B, _CH), jnp.int32),
    ],
    scratch_types=[
        pltpu.VMEM((_L,), jnp.float32),
        pltpu.VMEM((_L,), jnp.float32),
        pltpu.VMEM((_CH,), jnp.float32),
        pltpu.VMEM((_CH,), jnp.float32),
        pltpu.VMEM((_CH,), jnp.int32),
        pltpu.VMEM((_CH,), jnp.int32),
    ],
)(_body)


@jax.jit
def kernel(span_start_logits, span_end_logits):
    score, i_idx, j_idx = _sc_call(span_start_logits, span_end_logits)
    return score[:, 0], i_idx[:, 0], j_idx[:, 0]


# speedup vs baseline: 1.1226x; 1.1226x over previous
import functools

import jax
import jax.numpy as jnp
import numpy as np
from jax import lax
from jax.experimental import pallas as pl
from jax.experimental.pallas import tpu as pltpu
from jax.experimental.pallas import tpu_sc as plsc

_B, _L = 16, 2048
_CH = 16
_SEG = _L // _CH
_IMAX = np.int32(2147483647)
_NEG_INF = np.float32(-np.inf)


def _body(start_hbm, end_hbm, out_hbm, start_v, end_v, seg_s, out_s,
          sem1, sem2):
    row = lax.axis_index("s")

    cp1 = pltpu.async_copy(start_hbm.at[row], start_v, sem1)
    cp2 = pltpu.async_copy(end_hbm.at[row], end_v, sem2)
    cp2.wait()

    lane = lax.iota(jnp.int32, _CH)
    lanebase = lane * _SEG

    def segmax(k, m):
        e = plsc.load_gather(end_v, [lanebase + k])
        return jnp.maximum(m, e)

    sm = lax.fori_loop(0, _SEG, segmax,
                       jnp.zeros((_CH,), jnp.float32) + _NEG_INF,
                       unroll=8)

    segsuf = lax.rev(plsc.cummax(lax.rev(sm, (0,))), (0,))
    seg_s[...] = segsuf
    h = plsc.load_gather(seg_s, [jnp.minimum(lane + 1, _CH - 1)])
    h = jnp.where(lane == _CH - 1, _NEG_INF, h)

    cp1.wait()

    def bwd(k, state):
        mm, bestv, besti = state
        kk = _SEG - 1 - k
        e = plsc.load_gather(end_v, [lanebase + kk])
        st = plsc.load_gather(start_v, [lanebase + kk])
        mm = jnp.maximum(mm, e)
        c = st + jnp.maximum(mm, h)
        upd = c >= bestv
        bestv = jnp.where(upd, c, bestv)
        besti = jnp.where(upd, lanebase + kk, besti)
        return mm, bestv, besti

    init = (jnp.zeros((_CH,), jnp.float32) + _NEG_INF,
            jnp.zeros((_CH,), jnp.float32) + _NEG_INF,
            jnp.zeros((_CH,), jnp.int32))
    _, bestv, besti = lax.fori_loop(0, _SEG, bwd, init, unroll=8)

    best = jnp.max(bestv)
    istar = jnp.min(jnp.where(bestv == best, besti, _IMAX))
    lstar = istar // _SEG

    wacc = jnp.zeros((_CH,), jnp.float32) + _NEG_INF
    sbase = lstar * _SEG
    for t in range(_SEG // _CH):
        base = pl.multiple_of(sbase + t * _CH, _CH)
        e = end_v[pl.ds(base, _CH)]
        idxv = lane + base
        wacc = jnp.maximum(wacc, jnp.where(idxv >= istar, e, _NEG_INF))
    wval = jnp.max(wacc)
    hval = jnp.max(jnp.where(lane == lstar, h, _NEG_INF))
    target = jnp.maximum(wval, hval)
    inseg = wval >= hval

    lnext = jnp.min(jnp.where((lane > lstar) & (sm == target), lane, _IMAX))
    lsel = jnp.where(inseg, lstar, lnext)
    jlo = jnp.where(inseg, istar, lsel * _SEG)
    jacc = jnp.zeros((_CH,), jnp.int32) + _IMAX
    jbase = lsel * _SEG
    for t in range(_SEG // _CH):
        base = pl.multiple_of(jbase + t * _CH, _CH)
        e = end_v[pl.ds(base, _CH)]
        idxv = lane + base
        m = (idxv >= jlo) & (e == target)
        jacc = jnp.minimum(jacc, jnp.where(m, idxv, _IMAX))
    jstar = jnp.min(jacc)

    zf = jnp.zeros((_CH,), jnp.float32)
    zi = jnp.zeros((_CH,), jnp.int32)
    out_s[pl.ds(0, _CH)] = zf + best
    out_s[pl.ds(_CH, _CH)] = plsc.bitcast(zi + istar, jnp.float32)
    out_s[pl.ds(2 * _CH, _CH)] = plsc.bitcast(zi + jstar, jnp.float32)
    pltpu.sync_copy(out_s, out_hbm.at[row])


_sc_call = functools.partial(
    pl.kernel,
    mesh=plsc.VectorSubcoreMesh(core_axis_name="c", subcore_axis_name="s",
                                num_cores=1),
    compiler_params=pltpu.CompilerParams(needs_layout_passes=False),
    out_type=jax.ShapeDtypeStruct((_B, 3 * _CH), jnp.float32),
    scratch_types=[
        pltpu.VMEM((_L,), jnp.float32),
        pltpu.VMEM((_L,), jnp.float32),
        pltpu.VMEM((_CH,), jnp.float32),
        pltpu.VMEM((3 * _CH,), jnp.float32),
        pltpu.SemaphoreType.DMA,
        pltpu.SemaphoreType.DMA,
    ],
)(_body)


@jax.jit
def kernel(span_start_logits, span_end_logits):
    out = _sc_call(span_start_logits, span_end_logits)
    score = out[:, 0]
    i_idx = lax.bitcast_convert_type(out[:, _CH], jnp.int32)
    j_idx = lax.bitcast_convert_type(out[:, 2 * _CH], jnp.int32)
    return score, i_idx, j_idx
